# baseline (speedup 1.0000x reference)
import functools

import jax
import jax.numpy as jnp
from jax import lax
from jax.experimental import pallas as pl
from jax.experimental.pallas import tpu as pltpu
from jax.experimental.pallas import tpu_sc as plsc

T = 2048
H = 1024
I = 512
E = 8
K = 2
B = 256
NUM_BLOCKS = 24
R_PAD = NUM_BLOCKS * B
NC, NS = 2, 16
NW = NC * NS
ROWS_PER_W = R_PAD // NW
TOKS_PER_W = T // NW

_SC_MESH = plsc.VectorSubcoreMesh(core_axis_name="c", subcore_axis_name="s")


def _worker_id():
    return lax.axis_index("s") * NC + lax.axis_index("c")


_DR = 48
_DN = ROWS_PER_W // _DR


@functools.partial(
    pl.kernel,
    out_type=jax.ShapeDtypeStruct((R_PAD, H), jnp.float32),
    mesh=_SC_MESH,
    name="sc_dispatch_gather",
    scratch_types=[
        pltpu.VMEM((ROWS_PER_W,), jnp.int32),
        pltpu.VMEM((_DR, H), jnp.float32),
        pltpu.VMEM((_DR, H), jnp.float32),
        pltpu.SemaphoreType.DMA,
        pltpu.SemaphoreType.DMA,
        pltpu.SemaphoreType.DMA,
        pltpu.SemaphoreType.DMA,
    ],
)
def _dispatch(hid_hbm, rid_hbm, xs_hbm, idx_v, row0, row1, g0, g1, s0, s1):
    base = _worker_id() * ROWS_PER_W
    pltpu.sync_copy(rid_hbm.at[pl.ds(base, ROWS_PER_W)], idx_v)
    rows = (row0, row1)
    gsem = (g0, g1)
    ssem = (s0, s1)

    def _gather(r, b):
        return pltpu.async_copy(
            hid_hbm.at[idx_v.at[pl.ds(r * _DR, _DR)]], rows[b], gsem[b])

    scatters = [None, None]
    cur = _gather(0, 0)
    for r in range(_DN):
        b = r % 2
        nb = 1 - b
        nxt = None
        if r + 1 < _DN:
            if scatters[nb] is not None:
                scatters[nb].wait()
            nxt = _gather(r + 1, nb)
        cur.wait()
        scatters[b] = pltpu.async_copy(
            rows[b], xs_hbm.at[pl.ds(base + r * _DR, _DR)], ssem[b])
        cur = nxt
    scatters[0].wait()
    scatters[1].wait()


def _ffn_body(ge_ref, x_ref, gu_ref, dp_ref, w_ref, o_ref):
    del ge_ref
    x = x_ref[...].astype(jnp.bfloat16)
    w1 = gu_ref[0]
    xw = lax.dot_general(x, w1, (((1,), (1,)), ((), ())),
                         preferred_element_type=jnp.float32)
    gate = xw[:, :I]
    up = xw[:, I:]
    h = (gate * lax.logistic(gate) * up).astype(jnp.bfloat16)
    w2 = dp_ref[0]
    out = lax.dot_general(h, w2, (((1,), (1,)), ((), ())),
                          preferred_element_type=jnp.float32)
    o_ref[...] = out * w_ref[0, 0, :][:, None]


_ffn = pl.pallas_call(
    _ffn_body,
    grid_spec=pltpu.PrefetchScalarGridSpec(
        num_scalar_prefetch=1,
        grid=(NUM_BLOCKS,),
        in_specs=[
            pl.BlockSpec((B, H), lambda b, ge: (b, 0)),
            pl.BlockSpec((1, 2 * I, H), lambda b, ge: (ge[b], 0, 0)),
            pl.BlockSpec((1, H, I), lambda b, ge: (ge[b], 0, 0)),
            pl.BlockSpec((1, 1, B), lambda b, ge: (b, 0, 0)),
        ],
        out_specs=pl.BlockSpec((B, H), lambda b, ge: (b, 0)),
    ),
    out_shape=jax.ShapeDtypeStruct((R_PAD, H), jnp.float32),
)


_CR = 16
_CN = TOKS_PER_W // _CR


@functools.partial(
    pl.kernel,
    out_type=(
        jax.ShapeDtypeStruct((T, H), jnp.float32),
        jax.ShapeDtypeStruct((T, H), jnp.float32),
    ),
    mesh=_SC_MESH,
    name="sc_combine_gather",
    scratch_types=[
        pltpu.VMEM((TOKS_PER_W,), jnp.int32),
        pltpu.VMEM((TOKS_PER_W,), jnp.int32),
        pltpu.VMEM((_CR, H), jnp.float32),
        pltpu.VMEM((_CR, H), jnp.float32),
        pltpu.VMEM((_CR, H), jnp.float32),
        pltpu.VMEM((_CR, H), jnp.float32),
        pltpu.SemaphoreType.DMA,
        pltpu.SemaphoreType.DMA,
        pltpu.SemaphoreType.DMA,
        pltpu.SemaphoreType.DMA,
        pltpu.SemaphoreType.DMA,
        pltpu.SemaphoreType.DMA,
    ],
)
def _combine_gather(os_hbm, pe_hbm, po_hbm, oute_hbm, outo_hbm,
                    ie_v, io_v, be0, bo0, be1, bo1,
                    ge0, go0, ge1, go1, ss0, ss1):
    base = _worker_id() * TOKS_PER_W
    pltpu.sync_copy(pe_hbm.at[pl.ds(base, TOKS_PER_W)], ie_v)
    pltpu.sync_copy(po_hbm.at[pl.ds(base, TOKS_PER_W)], io_v)
    bes = (be0, be1)
    bos = (bo0, bo1)
    gesem = (ge0, ge1)
    gosem = (go0, go1)
    ssem = (ss0, ss1)

    def _gather(r, b):
        ce = pltpu.async_copy(
            os_hbm.at[ie_v.at[pl.ds(r * _CR, _CR)]], bes[b], gesem[b])
        co = pltpu.async_copy(
            os_hbm.at[io_v.at[pl.ds(r * _CR, _CR)]], bos[b], gosem[b])
        return ce, co

    scatters = [None, None]
    cur = _gather(0, 0)
    for r in range(_CN):
        b = r % 2
        nb = 1 - b
        nxt = None
        if r + 1 < _CN:
            if scatters[nb] is not None:
                scatters[nb][0].wait()
                scatters[nb][1].wait()
            nxt = _gather(r + 1, nb)
        cur[0].wait()
        cur[1].wait()
        off = base + r * _CR
        scatters[b] = (
            pltpu.async_copy(bes[b], oute_hbm.at[pl.ds(off, _CR)], ssem[b]),
            pltpu.async_copy(bos[b], outo_hbm.at[pl.ds(off, _CR)], ssem[b]),
        )
        cur = nxt
    for sc in scatters:
        if sc is not None:
            sc[0].wait()
            sc[1].wait()


def _add_body(a_ref, b_ref, o_ref):
    o_ref[...] = a_ref[...] + b_ref[...]


_final_add = pl.pallas_call(
    _add_body,
    grid=(T // 512,),
    in_specs=[
        pl.BlockSpec((512, H), lambda i: (i, 0)),
        pl.BlockSpec((512, H), lambda i: (i, 0)),
    ],
    out_specs=pl.BlockSpec((512, H), lambda i: (i, 0)),
    out_shape=jax.ShapeDtypeStruct((T, H), jnp.float32),
)


def kernel(hidden_states, top_k_indices, top_k_weights, gate_up_proj, down_proj):
    e_flat = top_k_indices.reshape(-1).astype(jnp.int32)
    w_flat = top_k_weights.reshape(-1)
    onehot = (e_flat[:, None] == jnp.arange(E, dtype=jnp.int32)[None, :]
              ).astype(jnp.int32)
    ranks_inc = jnp.cumsum(onehot, axis=0)
    counts = ranks_inc[-1]
    rank = jnp.take_along_axis(ranks_inc, e_flat[:, None], axis=1)[:, 0] - 1
    padded = ((counts + B - 1) // B) * B
    pend = jnp.cumsum(padded)
    pstart = pend - padded
    pos_flat = (pstart[e_flat] + rank).astype(jnp.int32)
    block_expert = jnp.minimum(
        jnp.searchsorted(pend, jnp.arange(NUM_BLOCKS, dtype=jnp.int32) * B,
                         side="right"),
        E - 1,
    ).astype(jnp.int32)
    tok_flat = jnp.arange(T * K, dtype=jnp.int32) // K
    pad_init = jnp.arange(R_PAD, dtype=jnp.int32) % T
    row_ids = pad_init.at[pos_flat].set(
        tok_flat, mode="promise_in_bounds", unique_indices=True)
    w_sorted = jnp.zeros((R_PAD,), jnp.float32).at[pos_flat].set(
        w_flat, mode="promise_in_bounds", unique_indices=True)
    pos_tk = pos_flat.reshape(T, K)
    pos_e = pos_tk[:, 0]
    pos_o = pos_tk[:, 1]

    x_sorted = _dispatch(hidden_states, row_ids)
    out_sorted = _ffn(block_expert, x_sorted,
                      gate_up_proj.astype(jnp.bfloat16),
                      down_proj.astype(jnp.bfloat16),
                      w_sorted.reshape(NUM_BLOCKS, 1, B))
    out_e_rows, out_o_rows = _combine_gather(out_sorted, pos_e, pos_o)
    return _final_add(out_e_rows, out_o_rows)


# speedup vs baseline: 1.2257x; 1.2257x over previous
import functools

import jax
import jax.numpy as jnp
from jax import lax
from jax.experimental import pallas as pl
from jax.experimental.pallas import tpu as pltpu
from jax.experimental.pallas import tpu_sc as plsc

T = 2048
H = 1024
I = 512
E = 8
K = 2
B = 256
NUM_BLOCKS = 23
R_PAD = NUM_BLOCKS * B
NC, NS = 2, 16
NW = NC * NS
ROWS_PER_W = R_PAD // NW
TOKS_PER_W = T // NW

_SC_MESH = plsc.VectorSubcoreMesh(core_axis_name="c", subcore_axis_name="s")


def _worker_id():
    return lax.axis_index("s") * NC + lax.axis_index("c")


_DCH = (48, 48, 48, 40)
_DOFF = (0, 48, 96, 144)
_DN = len(_DCH)


@functools.partial(
    pl.kernel,
    out_type=jax.ShapeDtypeStruct((R_PAD, H), jnp.float32),
    mesh=_SC_MESH,
    name="sc_dispatch_gather",
    scratch_types=[
        pltpu.VMEM((ROWS_PER_W,), jnp.int32),
        pltpu.VMEM((_DCH[0], H), jnp.float32),
        pltpu.VMEM((_DCH[0], H), jnp.float32),
        pltpu.SemaphoreType.DMA,
        pltpu.SemaphoreType.DMA,
        pltpu.SemaphoreType.DMA,
        pltpu.SemaphoreType.DMA,
    ],
)
def _dispatch(hid_hbm, rid_hbm, xs_hbm, idx_v, row0, row1, g0, g1, s0, s1):
    base = _worker_id() * ROWS_PER_W
    pltpu.sync_copy(rid_hbm.at[pl.ds(base, ROWS_PER_W)], idx_v)
    rows = (row0, row1)
    gsem = (g0, g1)
    ssem = (s0, s1)

    def _gather(r, b):
        n = _DCH[r]
        return pltpu.async_copy(
            hid_hbm.at[idx_v.at[pl.ds(_DOFF[r], n)]],
            rows[b].at[pl.ds(0, n)], gsem[b])

    scatters = [None, None]
    cur = _gather(0, 0)
    for r in range(_DN):
        b = r % 2
        nb = 1 - b
        nxt = None
        if r + 1 < _DN:
            if scatters[nb] is not None:
                scatters[nb].wait()
            nxt = _gather(r + 1, nb)
        cur.wait()
        scatters[b] = pltpu.async_copy(
            rows[b].at[pl.ds(0, _DCH[r])],
            xs_hbm.at[pl.ds(base + _DOFF[r], _DCH[r])], ssem[b])
        cur = nxt
    scatters[0].wait()
    scatters[1].wait()


def _ffn_body(ge_ref, x_ref, gu_ref, dp_ref, w_ref, o_ref):
    del ge_ref
    x = x_ref[...]
    w1 = gu_ref[0]
    xw = lax.dot_general(x, w1, (((1,), (1,)), ((), ())),
                         preferred_element_type=jnp.float32)
    gate = xw[:, :I]
    up = xw[:, I:]
    h = gate * lax.logistic(gate) * up
    w2 = dp_ref[0]
    out = lax.dot_general(h, w2, (((1,), (1,)), ((), ())),
                          preferred_element_type=jnp.float32)
    o_ref[...] = out * w_ref[0, 0, :][:, None]


_ffn = pl.pallas_call(
    _ffn_body,
    grid_spec=pltpu.PrefetchScalarGridSpec(
        num_scalar_prefetch=1,
        grid=(NUM_BLOCKS,),
        in_specs=[
            pl.BlockSpec((B, H), lambda b, ge: (b, 0)),
            pl.BlockSpec((1, 2 * I, H), lambda b, ge: (ge[b], 0, 0)),
            pl.BlockSpec((1, H, I), lambda b, ge: (ge[b], 0, 0)),
            pl.BlockSpec((1, 1, B), lambda b, ge: (b, 0, 0)),
        ],
        out_specs=pl.BlockSpec((B, H), lambda b, ge: (b, 0)),
    ),
    out_shape=jax.ShapeDtypeStruct((R_PAD, H), jnp.float32),
)


_CR = 16
_CN = TOKS_PER_W // _CR


@functools.partial(
    pl.kernel,
    out_type=(
        jax.ShapeDtypeStruct((T, H), jnp.float32),
        jax.ShapeDtypeStruct((T, H), jnp.float32),
    ),
    mesh=_SC_MESH,
    name="sc_combine_gather",
    scratch_types=[
        pltpu.VMEM((TOKS_PER_W,), jnp.int32),
        pltpu.VMEM((TOKS_PER_W,), jnp.int32),
        pltpu.VMEM((_CR, H), jnp.float32),
        pltpu.VMEM((_CR, H), jnp.float32),
        pltpu.VMEM((_CR, H), jnp.float32),
        pltpu.VMEM((_CR, H), jnp.float32),
        pltpu.SemaphoreType.DMA,
        pltpu.SemaphoreType.DMA,
        pltpu.SemaphoreType.DMA,
        pltpu.SemaphoreType.DMA,
        pltpu.SemaphoreType.DMA,
        pltpu.SemaphoreType.DMA,
    ],
)
def _combine_gather(os_hbm, pe_hbm, po_hbm, oute_hbm, outo_hbm,
                    ie_v, io_v, be0, bo0, be1, bo1,
                    ge0, go0, ge1, go1, ss0, ss1):
    base = _worker_id() * TOKS_PER_W
    pltpu.sync_copy(pe_hbm.at[pl.ds(base, TOKS_PER_W)], ie_v)
    pltpu.sync_copy(po_hbm.at[pl.ds(base, TOKS_PER_W)], io_v)
    bes = (be0, be1)
    bos = (bo0, bo1)
    gesem = (ge0, ge1)
    gosem = (go0, go1)
    ssem = (ss0, ss1)

    def _gather(r, b):
        ce = pltpu.async_copy(
            os_hbm.at[ie_v.at[pl.ds(r * _CR, _CR)]], bes[b], gesem[b])
        co = pltpu.async_copy(
            os_hbm.at[io_v.at[pl.ds(r * _CR, _CR)]], bos[b], gosem[b])
        return ce, co

    scatters = [None, None]
    cur = _gather(0, 0)
    for r in range(_CN):
        b = r % 2
        nb = 1 - b
        nxt = None
        if r + 1 < _CN:
            if scatters[nb] is not None:
                scatters[nb][0].wait()
                scatters[nb][1].wait()
            nxt = _gather(r + 1, nb)
        cur[0].wait()
        cur[1].wait()
        off = base + r * _CR
        scatters[b] = (
            pltpu.async_copy(bes[b], oute_hbm.at[pl.ds(off, _CR)], ssem[b]),
            pltpu.async_copy(bos[b], outo_hbm.at[pl.ds(off, _CR)], ssem[b]),
        )
        cur = nxt
    for sc in scatters:
        if sc is not None:
            sc[0].wait()
            sc[1].wait()


def _add_body(a_ref, b_ref, o_ref):
    o_ref[...] = a_ref[...] + b_ref[...]


_final_add = pl.pallas_call(
    _add_body,
    grid=(T // 512,),
    in_specs=[
        pl.BlockSpec((512, H), lambda i: (i, 0)),
        pl.BlockSpec((512, H), lambda i: (i, 0)),
    ],
    out_specs=pl.BlockSpec((512, H), lambda i: (i, 0)),
    out_shape=jax.ShapeDtypeStruct((T, H), jnp.float32),
)


def kernel(hidden_states, top_k_indices, top_k_weights, gate_up_proj, down_proj):
    e_flat = top_k_indices.reshape(-1).astype(jnp.int32)
    w_flat = top_k_weights.reshape(-1)
    onehot = (e_flat[:, None] == jnp.arange(E, dtype=jnp.int32)[None, :]
              ).astype(jnp.int32)
    ranks_inc = jnp.cumsum(onehot, axis=0)
    counts = ranks_inc[-1]
    rank = jnp.take_along_axis(ranks_inc, e_flat[:, None], axis=1)[:, 0] - 1
    padded = ((counts + B - 1) // B) * B
    pend = jnp.cumsum(padded)
    pstart = pend - padded
    pos_flat = (pstart[e_flat] + rank).astype(jnp.int32)
    block_expert = jnp.minimum(
        jnp.searchsorted(pend, jnp.arange(NUM_BLOCKS, dtype=jnp.int32) * B,
                         side="right"),
        E - 1,
    ).astype(jnp.int32)
    tok_flat = jnp.arange(T * K, dtype=jnp.int32) // K
    pad_init = jnp.arange(R_PAD, dtype=jnp.int32) % T
    row_ids = pad_init.at[pos_flat].set(
        tok_flat, mode="promise_in_bounds", unique_indices=True)
    w_sorted = jnp.zeros((R_PAD,), jnp.float32).at[pos_flat].set(
        w_flat, mode="promise_in_bounds", unique_indices=True)
    pos_tk = pos_flat.reshape(T, K)
    pos_e = pos_tk[:, 0]
    pos_o = pos_tk[:, 1]

    x_sorted = _dispatch(hidden_states, row_ids)
    out_sorted = _ffn(block_expert, x_sorted, gate_up_proj, down_proj,
                      w_sorted.reshape(NUM_BLOCKS, 1, B))
    out_e_rows, out_o_rows = _combine_gather(out_sorted, pos_e, pos_o)
    return _final_add(out_e_rows, out_o_rows)


# speedup vs baseline: 1.2938x; 1.0556x over previous
import functools

import jax
import jax.numpy as jnp
from jax import lax
from jax.experimental import pallas as pl
from jax.experimental.pallas import tpu as pltpu
from jax.experimental.pallas import tpu_sc as plsc

T = 2048
H = 1024
I = 512
E = 8
K = 2
B = 256
NUM_BLOCKS = 23
R_PAD = NUM_BLOCKS * B
NC, NS = 2, 16
NW = NC * NS
ROWS_PER_W = R_PAD // NW
TOKS_PER_W = T // NW

_SC_MESH = plsc.VectorSubcoreMesh(core_axis_name="c", subcore_axis_name="s")


def _worker_id():
    return lax.axis_index("s") * NC + lax.axis_index("c")


_DCH = (48, 48, 48, 40)
_DOFF = (0, 48, 96, 144)
_DN = len(_DCH)


@functools.partial(
    pl.kernel,
    out_type=jax.ShapeDtypeStruct((R_PAD, H), jnp.float32),
    mesh=_SC_MESH,
    name="sc_dispatch_gather",
    scratch_types=[
        pltpu.VMEM((ROWS_PER_W,), jnp.int32),
        pltpu.VMEM((_DCH[0], H), jnp.float32),
        pltpu.VMEM((_DCH[0], H), jnp.float32),
        pltpu.SemaphoreType.DMA,
        pltpu.SemaphoreType.DMA,
        pltpu.SemaphoreType.DMA,
        pltpu.SemaphoreType.DMA,
    ],
)
def _dispatch(hid_hbm, rid_hbm, xs_hbm, idx_v, row0, row1, g0, g1, s0, s1):
    base = _worker_id() * ROWS_PER_W
    pltpu.sync_copy(rid_hbm.at[pl.ds(base, ROWS_PER_W)], idx_v)
    rows = (row0, row1)
    gsem = (g0, g1)
    ssem = (s0, s1)

    def _gather(r, b):
        n = _DCH[r]
        return pltpu.async_copy(
            hid_hbm.at[idx_v.at[pl.ds(_DOFF[r], n)]],
            rows[b].at[pl.ds(0, n)], gsem[b])

    scatters = [None, None]
    cur = _gather(0, 0)
    for r in range(_DN):
        b = r % 2
        nb = 1 - b
        nxt = None
        if r + 1 < _DN:
            if scatters[nb] is not None:
                scatters[nb].wait()
            nxt = _gather(r + 1, nb)
        cur.wait()
        scatters[b] = pltpu.async_copy(
            rows[b].at[pl.ds(0, _DCH[r])],
            xs_hbm.at[pl.ds(base + _DOFF[r], _DCH[r])], ssem[b])
        cur = nxt
    scatters[0].wait()
    scatters[1].wait()


def _ffn_body(ge_ref, x_ref, gu_ref, dp_ref, w_ref, o_ref):
    del ge_ref
    x = x_ref[...]
    w1 = gu_ref[0]
    xw = lax.dot_general(x, w1, (((1,), (1,)), ((), ())),
                         preferred_element_type=jnp.float32)
    gate = xw[:, :I]
    up = xw[:, I:]
    h = gate * lax.logistic(gate) * up
    w2 = dp_ref[0]
    out = lax.dot_general(h, w2, (((1,), (1,)), ((), ())),
                          preferred_element_type=jnp.float32)
    o_ref[...] = out * w_ref[0, 0, :][:, None]


_ffn = pl.pallas_call(
    _ffn_body,
    grid_spec=pltpu.PrefetchScalarGridSpec(
        num_scalar_prefetch=1,
        grid=(NUM_BLOCKS,),
        in_specs=[
            pl.BlockSpec((B, H), lambda b, ge: (b, 0)),
            pl.BlockSpec((1, 2 * I, H), lambda b, ge: (ge[b], 0, 0)),
            pl.BlockSpec((1, H, I), lambda b, ge: (ge[b], 0, 0)),
            pl.BlockSpec((1, 1, B), lambda b, ge: (b, 0, 0)),
        ],
        out_specs=pl.BlockSpec((B, H), lambda b, ge: (b, 0)),
    ),
    out_shape=jax.ShapeDtypeStruct((R_PAD, H), jnp.float32),
)


_CR = 16
_CN = TOKS_PER_W // _CR


@functools.partial(
    pl.kernel,
    out_type=jax.ShapeDtypeStruct((T, H), jnp.float32),
    mesh=_SC_MESH,
    name="sc_combine_gather",
    scratch_types=[
        pltpu.VMEM((TOKS_PER_W,), jnp.int32),
        pltpu.VMEM((TOKS_PER_W,), jnp.int32),
        pltpu.VMEM((_CR, H), jnp.float32),
        pltpu.VMEM((_CR, H), jnp.float32),
        pltpu.VMEM((_CR, H), jnp.float32),
        pltpu.VMEM((_CR, H), jnp.float32),
        pltpu.SemaphoreType.DMA,
        pltpu.SemaphoreType.DMA,
        pltpu.SemaphoreType.DMA,
        pltpu.SemaphoreType.DMA,
        pltpu.SemaphoreType.DMA,
        pltpu.SemaphoreType.DMA,
    ],
)
def _combine_gather(os_hbm, pe_hbm, po_hbm, out_hbm,
                    ie_v, io_v, be0, bo0, be1, bo1,
                    ge0, go0, ge1, go1, ss0, ss1):
    base = _worker_id() * TOKS_PER_W
    pltpu.sync_copy(pe_hbm.at[pl.ds(base, TOKS_PER_W)], ie_v)
    pltpu.sync_copy(po_hbm.at[pl.ds(base, TOKS_PER_W)], io_v)
    bes = (be0, be1)
    bos = (bo0, bo1)
    gesem = (ge0, ge1)
    gosem = (go0, go1)
    ssem = (ss0, ss1)

    def _gather(r, b):
        ce = pltpu.async_copy(
            os_hbm.at[ie_v.at[pl.ds(r * _CR, _CR)]], bes[b], gesem[b])
        co = pltpu.async_copy(
            os_hbm.at[io_v.at[pl.ds(r * _CR, _CR)]], bos[b], gosem[b])
        return ce, co

    scatters = [None, None]
    cur = _gather(0, 0)
    for r in range(_CN):
        b = r % 2
        nb = 1 - b
        nxt = None
        if r + 1 < _CN:
            if scatters[nb] is not None:
                scatters[nb].wait()
            nxt = _gather(r + 1, nb)
        cur[0].wait()
        cur[1].wait()
        be_v = bes[b]
        bo_v = bos[b]

        def _add_row(i, _):
            for s in range(H // 16):
                be_v[i, pl.ds(s * 16, 16)] = (
                    be_v[i, pl.ds(s * 16, 16)] + bo_v[i, pl.ds(s * 16, 16)]
                )
            return 0

        lax.fori_loop(0, _CR, _add_row, 0)
        scatters[b] = pltpu.async_copy(
            bes[b], out_hbm.at[pl.ds(base + r * _CR, _CR)], ssem[b])
        cur = nxt
    for sc in scatters:
        if sc is not None:
            sc.wait()


def kernel(hidden_states, top_k_indices, top_k_weights, gate_up_proj, down_proj):
    e_flat = top_k_indices.reshape(-1).astype(jnp.int32)
    w_flat = top_k_weights.reshape(-1)
    onehot = (e_flat[:, None] == jnp.arange(E, dtype=jnp.int32)[None, :]
              ).astype(jnp.int32)
    ranks_inc = jnp.cumsum(onehot, axis=0)
    counts = ranks_inc[-1]
    rank = jnp.take_along_axis(ranks_inc, e_flat[:, None], axis=1)[:, 0] - 1
    padded = ((counts + B - 1) // B) * B
    pend = jnp.cumsum(padded)
    pstart = pend - padded
    pos_flat = (pstart[e_flat] + rank).astype(jnp.int32)
    block_expert = jnp.minimum(
        jnp.searchsorted(pend, jnp.arange(NUM_BLOCKS, dtype=jnp.int32) * B,
                         side="right"),
        E - 1,
    ).astype(jnp.int32)
    tok_flat = jnp.arange(T * K, dtype=jnp.int32) // K
    pad_init = jnp.arange(R_PAD, dtype=jnp.int32) % T
    row_ids = pad_init.at[pos_flat].set(
        tok_flat, mode="promise_in_bounds", unique_indices=True)
    w_sorted = jnp.zeros((R_PAD,), jnp.float32).at[pos_flat].set(
        w_flat, mode="promise_in_bounds", unique_indices=True)
    pos_tk = pos_flat.reshape(T, K)
    pos_e = pos_tk[:, 0]
    pos_o = pos_tk[:, 1]

    x_sorted = _dispatch(hidden_states, row_ids)
    out_sorted = _ffn(block_expert, x_sorted, gate_up_proj, down_proj,
                      w_sorted.reshape(NUM_BLOCKS, 1, B))
    return _combine_gather(out_sorted, pos_e, pos_o)


# speedup vs baseline: 1.4483x; 1.1194x over previous
import functools

import jax
import jax.numpy as jnp
from jax import lax
from jax.experimental import pallas as pl
from jax.experimental.pallas import tpu as pltpu
from jax.experimental.pallas import tpu_sc as plsc

T = 2048
H = 1024
I = 512
E = 8
K = 2
B = 256
NUM_BLOCKS = 23
R_PAD = NUM_BLOCKS * B
NC, NS = 2, 16
NW = NC * NS
ROWS_PER_W = R_PAD // NW
TOKS_PER_W = T // NW

_SC_MESH = plsc.VectorSubcoreMesh(core_axis_name="c", subcore_axis_name="s")


def _worker_id():
    return lax.axis_index("s") * NC + lax.axis_index("c")


@functools.partial(
    pl.kernel,
    out_type=jax.ShapeDtypeStruct((R_PAD, H), jnp.float32),
    mesh=_SC_MESH,
    name="sc_dispatch_scatter",
    scratch_types=[
        pltpu.VMEM((K, TOKS_PER_W), jnp.int32),
        pltpu.VMEM((TOKS_PER_W, H), jnp.float32),
        pltpu.SemaphoreType.DMA,
        pltpu.SemaphoreType.DMA,
    ],
)
def _dispatch(hid_hbm, pos3_hbm, xs_hbm, idx2_v, rows_v, s0, s1):
    wid = _worker_id()
    base = wid * TOKS_PER_W
    pltpu.sync_copy(pos3_hbm.at[wid], idx2_v)
    pltpu.sync_copy(hid_hbm.at[pl.ds(base, TOKS_PER_W)], rows_v)
    ce = pltpu.async_copy(rows_v, xs_hbm.at[idx2_v.at[0]], s0)
    co = pltpu.async_copy(rows_v, xs_hbm.at[idx2_v.at[1]], s1)
    ce.wait()
    co.wait()


def _ffn_body(ge_ref, x_ref, gu_ref, dp_ref, w_ref, o_ref):
    del ge_ref
    x = x_ref[...]
    w1 = gu_ref[0]
    xw = lax.dot_general(x, w1, (((1,), (1,)), ((), ())),
                         preferred_element_type=jnp.float32)
    gate = xw[:, :I]
    up = xw[:, I:]
    h = gate * lax.logistic(gate) * up
    w2 = dp_ref[0]
    out = lax.dot_general(h, w2, (((1,), (1,)), ((), ())),
                          preferred_element_type=jnp.float32)
    o_ref[...] = out * w_ref[0, 0, :][:, None]


_ffn = pl.pallas_call(
    _ffn_body,
    grid_spec=pltpu.PrefetchScalarGridSpec(
        num_scalar_prefetch=1,
        grid=(NUM_BLOCKS,),
        in_specs=[
            pl.BlockSpec((B, H), lambda b, ge: (b, 0)),
            pl.BlockSpec((1, 2 * I, H), lambda b, ge: (ge[b], 0, 0)),
            pl.BlockSpec((1, H, I), lambda b, ge: (ge[b], 0, 0)),
            pl.BlockSpec((1, 1, B), lambda b, ge: (b, 0, 0)),
        ],
        out_specs=pl.BlockSpec((B, H), lambda b, ge: (b, 0)),
    ),
    out_shape=jax.ShapeDtypeStruct((R_PAD, H), jnp.float32),
)


_CR = 16
_CN = TOKS_PER_W // _CR


@functools.partial(
    pl.kernel,
    out_type=jax.ShapeDtypeStruct((T, H), jnp.float32),
    mesh=_SC_MESH,
    name="sc_combine_gather",
    scratch_types=[
        pltpu.VMEM((TOKS_PER_W,), jnp.int32),
        pltpu.VMEM((TOKS_PER_W,), jnp.int32),
        pltpu.VMEM((_CR, H), jnp.float32),
        pltpu.VMEM((_CR, H), jnp.float32),
        pltpu.VMEM((_CR, H), jnp.float32),
        pltpu.VMEM((_CR, H), jnp.float32),
        pltpu.SemaphoreType.DMA,
        pltpu.SemaphoreType.DMA,
        pltpu.SemaphoreType.DMA,
        pltpu.SemaphoreType.DMA,
        pltpu.SemaphoreType.DMA,
        pltpu.SemaphoreType.DMA,
    ],
)
def _combine_gather(os_hbm, pe_hbm, po_hbm, out_hbm,
                    ie_v, io_v, be0, bo0, be1, bo1,
                    ge0, go0, ge1, go1, ss0, ss1):
    base = _worker_id() * TOKS_PER_W
    pltpu.sync_copy(pe_hbm.at[pl.ds(base, TOKS_PER_W)], ie_v)
    pltpu.sync_copy(po_hbm.at[pl.ds(base, TOKS_PER_W)], io_v)
    bes = (be0, be1)
    bos = (bo0, bo1)
    gesem = (ge0, ge1)
    gosem = (go0, go1)
    ssem = (ss0, ss1)

    def _gather(r, b):
        ce = pltpu.async_copy(
            os_hbm.at[ie_v.at[pl.ds(r * _CR, _CR)]], bes[b], gesem[b])
        co = pltpu.async_copy(
            os_hbm.at[io_v.at[pl.ds(r * _CR, _CR)]], bos[b], gosem[b])
        return ce, co

    scatters = [None, None]
    cur = _gather(0, 0)
    for r in range(_CN):
        b = r % 2
        nb = 1 - b
        nxt = None
        if r + 1 < _CN:
            if scatters[nb] is not None:
                scatters[nb].wait()
            nxt = _gather(r + 1, nb)
        cur[0].wait()
        cur[1].wait()
        be_v = bes[b]
        bo_v = bos[b]

        def _add_row(i, _):
            for s in range(H // 16):
                be_v[i, pl.ds(s * 16, 16)] = (
                    be_v[i, pl.ds(s * 16, 16)] + bo_v[i, pl.ds(s * 16, 16)]
                )
            return 0

        lax.fori_loop(0, _CR, _add_row, 0)
        scatters[b] = pltpu.async_copy(
            bes[b], out_hbm.at[pl.ds(base + r * _CR, _CR)], ssem[b])
        cur = nxt
    for sc in scatters:
        if sc is not None:
            sc.wait()


def kernel(hidden_states, top_k_indices, top_k_weights, gate_up_proj, down_proj):
    e_flat = top_k_indices.reshape(-1).astype(jnp.int32)
    w_flat = top_k_weights.reshape(-1)
    onehot = (e_flat[:, None] == jnp.arange(E, dtype=jnp.int32)[None, :]
              ).astype(jnp.int32)
    ranks_inc = jnp.cumsum(onehot, axis=0)
    counts = ranks_inc[-1]
    rank = jnp.take_along_axis(ranks_inc, e_flat[:, None], axis=1)[:, 0] - 1
    padded = ((counts + B - 1) // B) * B
    pend = jnp.cumsum(padded)
    pstart = pend - padded
    pos_flat = (pstart[e_flat] + rank).astype(jnp.int32)
    block_expert = jnp.minimum(
        jnp.searchsorted(pend, jnp.arange(NUM_BLOCKS, dtype=jnp.int32) * B,
                         side="right"),
        E - 1,
    ).astype(jnp.int32)
    w_sorted = jnp.zeros((R_PAD,), jnp.float32).at[pos_flat].set(
        w_flat, mode="promise_in_bounds", unique_indices=True)
    pos_tk = pos_flat.reshape(T, K)
    pos_e = pos_tk[:, 0]
    pos_o = pos_tk[:, 1]
    pos3 = jnp.stack([pos_e.reshape(NW, TOKS_PER_W),
                      pos_o.reshape(NW, TOKS_PER_W)], axis=1)

    x_sorted = _dispatch(hidden_states, pos3)
    out_sorted = _ffn(block_expert, x_sorted, gate_up_proj, down_proj,
                      w_sorted.reshape(NUM_BLOCKS, 1, B))
    return _combine_gather(out_sorted, pos_e, pos_o)


# speedup vs baseline: 1.5196x; 1.0493x over previous
import functools

import jax
import jax.numpy as jnp
from jax import lax
from jax.experimental import pallas as pl
from jax.experimental.pallas import tpu as pltpu
from jax.experimental.pallas import tpu_sc as plsc

T = 2048
H = 1024
I = 512
E = 8
K = 2
B = 256
NUM_BLOCKS = 23
R_PAD = NUM_BLOCKS * B
NC, NS = 2, 16
NW = NC * NS
ROWS_PER_W = R_PAD // NW
TOKS_PER_W = T // NW

_SC_MESH = plsc.VectorSubcoreMesh(core_axis_name="c", subcore_axis_name="s")


def _worker_id():
    return lax.axis_index("s") * NC + lax.axis_index("c")


@functools.partial(
    pl.kernel,
    out_type=jax.ShapeDtypeStruct((R_PAD, H), jnp.float32),
    mesh=_SC_MESH,
    name="sc_dispatch_scatter",
    scratch_types=[
        pltpu.VMEM((K, TOKS_PER_W), jnp.int32),
        pltpu.VMEM((TOKS_PER_W, H), jnp.float32),
        pltpu.SemaphoreType.DMA,
        pltpu.SemaphoreType.DMA,
    ],
)
def _dispatch(hid_hbm, pos3_hbm, xs_hbm, idx2_v, rows_v, s0, s1):
    wid = _worker_id()
    base = wid * TOKS_PER_W
    pltpu.sync_copy(pos3_hbm.at[wid], idx2_v)
    pltpu.sync_copy(hid_hbm.at[pl.ds(base, TOKS_PER_W)], rows_v)
    ce = pltpu.async_copy(rows_v, xs_hbm.at[idx2_v.at[0]], s0)
    co = pltpu.async_copy(rows_v, xs_hbm.at[idx2_v.at[1]], s1)
    ce.wait()
    co.wait()


def _ffn_body(ge_ref, x_ref, gu_ref, dp_ref, w_ref, o_ref):
    del ge_ref
    x = x_ref[...]
    w1 = gu_ref[0]
    xw = lax.dot_general(x, w1, (((1,), (1,)), ((), ())),
                         preferred_element_type=jnp.float32)
    gate = xw[:, :I]
    up = xw[:, I:]
    h = gate * lax.logistic(gate) * up
    w2 = dp_ref[0]
    out = lax.dot_general(h, w2, (((1,), (1,)), ((), ())),
                          preferred_element_type=jnp.float32)
    o_ref[...] = out * w_ref[0, 0, :][:, None]


_ffn = pl.pallas_call(
    _ffn_body,
    grid_spec=pltpu.PrefetchScalarGridSpec(
        num_scalar_prefetch=1,
        grid=(NUM_BLOCKS,),
        in_specs=[
            pl.BlockSpec((B, H), lambda b, ge: (b, 0)),
            pl.BlockSpec((1, 2 * I, H), lambda b, ge: (ge[b], 0, 0)),
            pl.BlockSpec((1, H, I), lambda b, ge: (ge[b], 0, 0)),
            pl.BlockSpec((1, 1, B), lambda b, ge: (b, 0, 0)),
        ],
        out_specs=pl.BlockSpec((B, H), lambda b, ge: (b, 0)),
    ),
    out_shape=jax.ShapeDtypeStruct((R_PAD, H), jnp.float32),
)


_CR = 16
_CN = TOKS_PER_W // _CR


@functools.partial(
    pl.kernel,
    out_type=jax.ShapeDtypeStruct((T, H), jnp.float32),
    mesh=_SC_MESH,
    name="sc_combine_gather",
    scratch_types=[
        pltpu.VMEM((TOKS_PER_W,), jnp.int32),
        pltpu.VMEM((TOKS_PER_W,), jnp.int32),
        pltpu.VMEM((_CR, H), jnp.float32),
        pltpu.VMEM((_CR, H), jnp.float32),
        pltpu.VMEM((_CR, H), jnp.float32),
        pltpu.VMEM((_CR, H), jnp.float32),
        pltpu.VMEM((_CR, H), jnp.float32),
        pltpu.VMEM((_CR, H), jnp.float32),
        pltpu.SemaphoreType.DMA,
        pltpu.SemaphoreType.DMA,
        pltpu.SemaphoreType.DMA,
        pltpu.SemaphoreType.DMA,
        pltpu.SemaphoreType.DMA,
        pltpu.SemaphoreType.DMA,
        pltpu.SemaphoreType.DMA,
        pltpu.SemaphoreType.DMA,
        pltpu.SemaphoreType.DMA,
    ],
)
def _combine_gather(os_hbm, pe_hbm, po_hbm, out_hbm,
                    ie_v, io_v, be0, bo0, be1, bo1, be2, bo2,
                    ge0, go0, ge1, go1, ge2, go2, ss0, ss1, ss2):
    base = _worker_id() * TOKS_PER_W
    pltpu.sync_copy(pe_hbm.at[pl.ds(base, TOKS_PER_W)], ie_v)
    pltpu.sync_copy(po_hbm.at[pl.ds(base, TOKS_PER_W)], io_v)
    bes = (be0, be1, be2)
    bos = (bo0, bo1, bo2)
    gesem = (ge0, ge1, ge2)
    gosem = (go0, go1, go2)
    ssem = (ss0, ss1, ss2)
    nbuf = 3

    def _gather(r):
        b = r % nbuf
        ce = pltpu.async_copy(
            os_hbm.at[ie_v.at[pl.ds(r * _CR, _CR)]], bes[b], gesem[b])
        co = pltpu.async_copy(
            os_hbm.at[io_v.at[pl.ds(r * _CR, _CR)]], bos[b], gosem[b])
        return ce, co

    gathers = [None] * _CN
    scatters = [None] * _CN
    for r in range(min(nbuf - 1, _CN)):
        gathers[r] = _gather(r)
    for r in range(_CN):
        b = r % nbuf
        pre = r + nbuf - 1
        if pre < _CN:
            if scatters[pre - nbuf] is not None:
                scatters[pre - nbuf].wait()
            gathers[pre] = _gather(pre)
        gathers[r][0].wait()
        gathers[r][1].wait()
        be_v = bes[b]
        bo_v = bos[b]

        def _add_row(i, _):
            for s in range(H // 16):
                be_v[i, pl.ds(s * 16, 16)] = (
                    be_v[i, pl.ds(s * 16, 16)] + bo_v[i, pl.ds(s * 16, 16)]
                )
            return 0

        lax.fori_loop(0, _CR, _add_row, 0)
        scatters[r] = pltpu.async_copy(
            bes[b], out_hbm.at[pl.ds(base + r * _CR, _CR)], ssem[b])
    for r in range(max(0, _CN - nbuf), _CN):
        scatters[r].wait()


def kernel(hidden_states, top_k_indices, top_k_weights, gate_up_proj, down_proj):
    e_flat = top_k_indices.reshape(-1).astype(jnp.int32)
    w_flat = top_k_weights.reshape(-1)
    onehot = (e_flat[:, None] == jnp.arange(E, dtype=jnp.int32)[None, :]
              ).astype(jnp.int32)
    ranks_inc = jnp.cumsum(onehot, axis=0)
    counts = ranks_inc[-1]
    rank = jnp.sum(ranks_inc * onehot, axis=1) - 1
    padded = ((counts + B - 1) // B) * B
    pend = jnp.cumsum(padded)
    pstart = pend - padded
    pos_flat = (pstart[e_flat] + rank).astype(jnp.int32)
    block_expert = jnp.minimum(
        jnp.searchsorted(pend, jnp.arange(NUM_BLOCKS, dtype=jnp.int32) * B,
                         side="right"),
        E - 1,
    ).astype(jnp.int32)
    w_sorted = jnp.zeros((R_PAD,), jnp.float32).at[pos_flat].set(
        w_flat, mode="promise_in_bounds", unique_indices=True)
    pos_tk = pos_flat.reshape(T, K)
    pos_e = pos_tk[:, 0]
    pos_o = pos_tk[:, 1]
    pos3 = jnp.stack([pos_e.reshape(NW, TOKS_PER_W),
                      pos_o.reshape(NW, TOKS_PER_W)], axis=1)

    x_sorted = _dispatch(hidden_states, pos3)
    out_sorted = _ffn(block_expert, x_sorted, gate_up_proj, down_proj,
                      w_sorted.reshape(NUM_BLOCKS, 1, B))
    return _combine_gather(out_sorted, pos_e, pos_o)


# speedup vs baseline: 1.5783x; 1.0386x over previous
import functools

import jax
import jax.numpy as jnp
from jax import lax
from jax.experimental import pallas as pl
from jax.experimental.pallas import tpu as pltpu
from jax.experimental.pallas import tpu_sc as plsc

T = 2048
H = 1024
I = 512
E = 8
K = 2
B = 256
NUM_BLOCKS = 23
R_PAD = NUM_BLOCKS * B
NC, NS = 2, 16
NW = NC * NS
ROWS_PER_W = R_PAD // NW
TOKS_PER_W = T // NW

_SC_MESH = plsc.VectorSubcoreMesh(core_axis_name="c", subcore_axis_name="s")


def _worker_id():
    return lax.axis_index("s") * NC + lax.axis_index("c")


@functools.partial(
    pl.kernel,
    out_type=(
        jax.ShapeDtypeStruct((R_PAD, H), jnp.float32),
        jax.ShapeDtypeStruct((R_PAD, 128), jnp.float32),
    ),
    mesh=_SC_MESH,
    name="sc_dispatch_scatter",
    scratch_types=[
        pltpu.VMEM((K, TOKS_PER_W), jnp.int32),
        pltpu.VMEM((TOKS_PER_W, H), jnp.float32),
        pltpu.VMEM((K, TOKS_PER_W, 128), jnp.float32),
        pltpu.SemaphoreType.DMA,
        pltpu.SemaphoreType.DMA,
        pltpu.SemaphoreType.DMA,
        pltpu.SemaphoreType.DMA,
    ],
)
def _dispatch(hid_hbm, pos3_hbm, w16_hbm, xs_hbm, ws_hbm,
              idx2_v, rows_v, wbuf_v, s0, s1, s2, s3):
    wid = _worker_id()
    base = wid * TOKS_PER_W
    pltpu.sync_copy(pos3_hbm.at[wid], idx2_v)
    pltpu.sync_copy(w16_hbm.at[wid], wbuf_v)
    pltpu.sync_copy(hid_hbm.at[pl.ds(base, TOKS_PER_W)], rows_v)
    ce = pltpu.async_copy(rows_v, xs_hbm.at[idx2_v.at[0]], s0)
    co = pltpu.async_copy(rows_v, xs_hbm.at[idx2_v.at[1]], s1)
    cwe = pltpu.async_copy(wbuf_v.at[0], ws_hbm.at[idx2_v.at[0]], s2)
    cwo = pltpu.async_copy(wbuf_v.at[1], ws_hbm.at[idx2_v.at[1]], s3)
    ce.wait()
    co.wait()
    cwe.wait()
    cwo.wait()


def _ffn_body(ge_ref, x_ref, gu_ref, dp_ref, w_ref, o_ref):
    del ge_ref
    x = x_ref[...]
    w1 = gu_ref[0]
    xw = lax.dot_general(x, w1, (((1,), (1,)), ((), ())),
                         preferred_element_type=jnp.float32)
    gate = xw[:, :I]
    up = xw[:, I:]
    h = gate * lax.logistic(gate) * up
    w2 = dp_ref[0]
    out = lax.dot_general(h, w2, (((1,), (1,)), ((), ())),
                          preferred_element_type=jnp.float32)
    o_ref[...] = out * w_ref[:, 0:1]


_ffn = pl.pallas_call(
    _ffn_body,
    grid_spec=pltpu.PrefetchScalarGridSpec(
        num_scalar_prefetch=1,
        grid=(NUM_BLOCKS,),
        in_specs=[
            pl.BlockSpec((B, H), lambda b, ge: (b, 0)),
            pl.BlockSpec((1, 2 * I, H), lambda b, ge: (ge[b], 0, 0)),
            pl.BlockSpec((1, H, I), lambda b, ge: (ge[b], 0, 0)),
            pl.BlockSpec((B, 128), lambda b, ge: (b, 0)),
        ],
        out_specs=pl.BlockSpec((B, H), lambda b, ge: (b, 0)),
    ),
    out_shape=jax.ShapeDtypeStruct((R_PAD, H), jnp.float32),
)


_CR = 16
_CN = TOKS_PER_W // _CR


@functools.partial(
    pl.kernel,
    out_type=jax.ShapeDtypeStruct((T, H), jnp.float32),
    mesh=_SC_MESH,
    name="sc_combine_gather",
    scratch_types=[
        pltpu.VMEM((TOKS_PER_W,), jnp.int32),
        pltpu.VMEM((TOKS_PER_W,), jnp.int32),
        pltpu.VMEM((_CR, H), jnp.float32),
        pltpu.VMEM((_CR, H), jnp.float32),
        pltpu.VMEM((_CR, H), jnp.float32),
        pltpu.VMEM((_CR, H), jnp.float32),
        pltpu.VMEM((_CR, H), jnp.float32),
        pltpu.VMEM((_CR, H), jnp.float32),
        pltpu.SemaphoreType.DMA,
        pltpu.SemaphoreType.DMA,
        pltpu.SemaphoreType.DMA,
        pltpu.SemaphoreType.DMA,
        pltpu.SemaphoreType.DMA,
        pltpu.SemaphoreType.DMA,
        pltpu.SemaphoreType.DMA,
        pltpu.SemaphoreType.DMA,
        pltpu.SemaphoreType.DMA,
    ],
)
def _combine_gather(os_hbm, pe_hbm, po_hbm, out_hbm,
                    ie_v, io_v, be0, bo0, be1, bo1, be2, bo2,
                    ge0, go0, ge1, go1, ge2, go2, ss0, ss1, ss2):
    base = _worker_id() * TOKS_PER_W
    pltpu.sync_copy(pe_hbm.at[pl.ds(base, TOKS_PER_W)], ie_v)
    pltpu.sync_copy(po_hbm.at[pl.ds(base, TOKS_PER_W)], io_v)
    bes = (be0, be1, be2)
    bos = (bo0, bo1, bo2)
    gesem = (ge0, ge1, ge2)
    gosem = (go0, go1, go2)
    ssem = (ss0, ss1, ss2)
    nbuf = 3

    def _gather(r):
        b = r % nbuf
        ce = pltpu.async_copy(
            os_hbm.at[ie_v.at[pl.ds(r * _CR, _CR)]], bes[b], gesem[b])
        co = pltpu.async_copy(
            os_hbm.at[io_v.at[pl.ds(r * _CR, _CR)]], bos[b], gosem[b])
        return ce, co

    gathers = [None] * _CN
    scatters = [None] * _CN
    for r in range(min(nbuf - 1, _CN)):
        gathers[r] = _gather(r)
    for r in range(_CN):
        b = r % nbuf
        pre = r + nbuf - 1
        if pre < _CN:
            if scatters[pre - nbuf] is not None:
                scatters[pre - nbuf].wait()
            gathers[pre] = _gather(pre)
        gathers[r][0].wait()
        gathers[r][1].wait()
        be_v = bes[b]
        bo_v = bos[b]

        def _add_row(i, _):
            for s in range(H // 16):
                be_v[i, pl.ds(s * 16, 16)] = (
                    be_v[i, pl.ds(s * 16, 16)] + bo_v[i, pl.ds(s * 16, 16)]
                )
            return 0

        lax.fori_loop(0, _CR, _add_row, 0)
        scatters[r] = pltpu.async_copy(
            bes[b], out_hbm.at[pl.ds(base + r * _CR, _CR)], ssem[b])
    for r in range(max(0, _CN - nbuf), _CN):
        scatters[r].wait()


def kernel(hidden_states, top_k_indices, top_k_weights, gate_up_proj, down_proj):
    e_flat = top_k_indices.reshape(-1).astype(jnp.int32)
    w_flat = top_k_weights.reshape(-1)
    onehot = (e_flat[:, None] == jnp.arange(E, dtype=jnp.int32)[None, :]
              ).astype(jnp.int32)
    ranks_inc = jnp.cumsum(onehot, axis=0)
    counts = ranks_inc[-1]
    rank = jnp.sum(ranks_inc * onehot, axis=1) - 1
    padded = ((counts + B - 1) // B) * B
    pend = jnp.cumsum(padded)
    pstart = pend - padded
    pos_flat = (pstart[e_flat] + rank).astype(jnp.int32)
    block_expert = jnp.minimum(
        jnp.searchsorted(pend, jnp.arange(NUM_BLOCKS, dtype=jnp.int32) * B,
                         side="right"),
        E - 1,
    ).astype(jnp.int32)
    pos_tk = pos_flat.reshape(T, K)
    pos_e = pos_tk[:, 0]
    pos_o = pos_tk[:, 1]
    pos3 = jnp.stack([pos_e.reshape(NW, TOKS_PER_W),
                      pos_o.reshape(NW, TOKS_PER_W)], axis=1)
    w3 = w_flat.reshape(NW, TOKS_PER_W, K).transpose(0, 2, 1)
    w16 = jnp.broadcast_to(w3[..., None], (NW, K, TOKS_PER_W, 128))

    x_sorted, w_rows = _dispatch(hidden_states, pos3, w16)
    out_sorted = _ffn(block_expert, x_sorted, gate_up_proj, down_proj,
                      w_rows)
    return _combine_gather(out_sorted, pos_e, pos_o)
